# Initial kernel scaffold; baseline (speedup 1.0000x reference)
import jax
import jax.numpy as jnp
from jax.experimental import pallas as pl


def kernel(x, e, edge_index, segment_ids, W_i, W_h, W_o, b_o, W_mlp):
    raise NotImplementedError("write your pallas kernel here")


# speedup vs baseline: 2.6483x; 2.6483x over previous
import functools

import jax
import jax.numpy as jnp
from jax import lax
from jax.experimental import pallas as pl
from jax.experimental.pallas import tpu as pltpu
from jax.experimental.pallas import tpu_sc as plsc

F32 = jnp.float32

_N = 10000
_E = 320000
_D = 128
_ATOM = 128
_NHALF = _N // 2

_NC = 2
_NS = 16
_NW = _NC * _NS
_PER_W = _E // _NW
_K = 80
_CH = _PER_W // _K
_NP = 10240
_RPT = _NP // _NS
_ZR = 128

_BLK = 2560
_NBLK = _E // _BLK


@functools.cache
def _sc_kernels():
    mesh = plsc.VectorSubcoreMesh(
        core_axis_name="c", subcore_axis_name="s", num_cores=_NC,
        num_subcores=_NS)

    @functools.partial(
        pl.kernel,
        out_type=jax.ShapeDtypeStruct((_E, _D), F32),
        mesh=mesh,
        scratch_types=[
            pltpu.VMEM((_CH, _K), jnp.int32),
            pltpu.VMEM((_K, _D), F32),
            pltpu.SemaphoreType.DMA,
        ],
    )
    def gather(tab_hbm, idx_hbm, out_hbm, idx_v, rows_v, sem):
        cid = lax.axis_index("c")
        sid = lax.axis_index("s")
        wid = sid * _NC + cid
        pltpu.sync_copy(idx_hbm.at[wid], idx_v)

        def body(ch, c):
            base = wid * _PER_W + ch * _K
            pltpu.async_copy(tab_hbm.at[idx_v.at[ch]], rows_v, sem).wait()
            pltpu.sync_copy(rows_v, out_hbm.at[pl.ds(base, _K)])
            return c

        lax.fori_loop(0, _CH, body, 0)

    @functools.partial(
        pl.kernel,
        out_type=jax.ShapeDtypeStruct((_NC, _NP, _D), F32),
        mesh=mesh,
        scratch_types=[
            pltpu.VMEM((_CH, _K), jnp.int32),
            pltpu.VMEM((_K, _D), F32),
            pltpu.VMEM((_ZR, _D), F32),
            pltpu.VMEM_SHARED((_NP, _D), F32),
        ],
    )
    def scatter(msg_hbm, dst_hbm, out_hbm, idx_v, rows_v, zbuf, acc):
        cid = lax.axis_index("c")
        sid = lax.axis_index("s")
        wid = sid * _NC + cid
        z16 = jnp.zeros((16,), F32)

        def zrow(i, c):
            for j in range(8):
                zbuf[i, pl.ds(j * 16, 16)] = z16
            return c

        lax.fori_loop(0, _ZR, zrow, 0)

        def zacc(k, c):
            pltpu.sync_copy(zbuf, acc.at[pl.ds(sid * _RPT + k * _ZR, _ZR)])
            return c

        lax.fori_loop(0, _RPT // _ZR, zacc, 0)
        plsc.subcore_barrier()

        pltpu.sync_copy(dst_hbm.at[wid], idx_v)

        def body(ch, c):
            base = wid * _PER_W + ch * _K
            pltpu.sync_copy(msg_hbm.at[pl.ds(base, _K)], rows_v)
            pltpu.sync_copy(rows_v, acc.at[idx_v.at[ch]], add=True)
            return c

        lax.fori_loop(0, _CH, body, 0)
        plsc.subcore_barrier()

        def drain(k, c):
            r = sid * _RPT + k * _ZR
            pltpu.sync_copy(acc.at[pl.ds(r, _ZR)], out_hbm.at[cid, pl.ds(r, _ZR)])
            return c

        lax.fori_loop(0, _RPT // _ZR, drain, 0)

    return gather, scatter


def _mm_tc(xx, ww):
    nb = 10

    def kk(x_ref, w_ref, o_ref):
        o_ref[...] = jnp.dot(x_ref[...], w_ref[...],
                             preferred_element_type=F32)

    return pl.pallas_call(
        kk,
        grid=(nb,),
        in_specs=[pl.BlockSpec((_N // nb, _D), lambda i: (i, 0)),
                  pl.BlockSpec((_D, _D), lambda i: (0, 0))],
        out_specs=pl.BlockSpec((_N // nb, _D), lambda i: (i, 0)),
        out_shape=jax.ShapeDtypeStruct((_N, _D), F32),
    )(xx, ww)


def _combine_tc(parts):
    nb = 10

    def kk(p_ref, o_ref):
        o_ref[...] = p_ref[0] + p_ref[1]

    return pl.pallas_call(
        kk,
        grid=(nb,),
        in_specs=[pl.BlockSpec((2, _NP // nb, _D), lambda i: (0, i, 0))],
        out_specs=pl.BlockSpec((_NP // nb, _D), lambda i: (i, 0)),
        out_shape=jax.ShapeDtypeStruct((_NP, _D), F32),
    )(parts)


def _init_tc(g0, e, wib):
    def kk(g_ref, e_ref, w_ref, inp_ref, m_ref):
        v = g_ref[...] + jnp.dot(e_ref[...], w_ref[...],
                                 preferred_element_type=F32)
        inp_ref[...] = v
        m_ref[...] = jnp.maximum(v, 0.0)

    return pl.pallas_call(
        kk,
        grid=(_NBLK,),
        in_specs=[pl.BlockSpec((_BLK, _D), lambda i: (i, 0)),
                  pl.BlockSpec((_BLK, 16), lambda i: (i, 0)),
                  pl.BlockSpec((16, _D), lambda i: (0, 0))],
        out_specs=[pl.BlockSpec((_BLK, _D), lambda i: (i, 0)),
                   pl.BlockSpec((_BLK, _D), lambda i: (i, 0))],
        out_shape=[jax.ShapeDtypeStruct((_E, _D), F32),
                   jax.ShapeDtypeStruct((_E, _D), F32)],
    )(g0, e, wib)


def _update_tc(inp, g, m, wh):
    def kk(inp_ref, g_ref, m_ref, w_ref, o_ref):
        mm = m_ref[...]
        up = jnp.roll(mm, -1, axis=0)
        dn = jnp.roll(mm, 1, axis=0)
        ridx = lax.broadcasted_iota(jnp.int32, (_BLK, _D), 0)
        msw = jnp.where(ridx % 2 == 0, up, dn)
        a = jnp.dot(g_ref[...] - msw, w_ref[...], preferred_element_type=F32)
        o_ref[...] = jnp.maximum(inp_ref[...] + a, 0.0)

    return pl.pallas_call(
        kk,
        grid=(_NBLK,),
        in_specs=[pl.BlockSpec((_BLK, _D), lambda i: (i, 0)),
                  pl.BlockSpec((_BLK, _D), lambda i: (i, 0)),
                  pl.BlockSpec((_BLK, _D), lambda i: (i, 0)),
                  pl.BlockSpec((_D, _D), lambda i: (0, 0))],
        out_specs=pl.BlockSpec((_BLK, _D), lambda i: (i, 0)),
        out_shape=jax.ShapeDtypeStruct((_E, _D), F32),
    )(inp, g, m, wh)


def _final_tc(x, parts, seg8, wo1, wo2, b8, wmlp):
    def kk(x_ref, p_ref, s_ref, wo1_ref, wo2_ref, b_ref, wm_ref, o_ref):
        es = p_ref[0, :_N, :] + p_ref[1, :_N, :]
        h = (jnp.dot(x_ref[...], wo1_ref[...], preferred_element_type=F32)
             + jnp.dot(es, wo2_ref[...], preferred_element_type=F32)
             + b_ref[0:1, :])
        h = jnp.maximum(h, 0.0)
        diff = h[_NHALF:, :] - h[:_NHALF, :]
        t = jnp.maximum(jnp.dot(diff, wm_ref[...],
                                preferred_element_type=F32), 0.0)
        seg = jnp.broadcast_to(s_ref[0:1, :], (128, _NHALF))
        oh = (seg == lax.broadcasted_iota(jnp.int32, (128, _NHALF), 0))
        o_ref[...] = jnp.dot(oh.astype(F32), t, preferred_element_type=F32)

    return pl.pallas_call(
        kk,
        out_shape=jax.ShapeDtypeStruct((128, _D), F32),
    )(x, parts, seg8, wo1, wo2, b8, wmlp)


def kernel(x, e, edge_index, segment_ids, W_i, W_h, W_o, b_o, W_mlp):
    src = edge_index[0]
    dst = edge_index[1]
    src3 = src.reshape(_NW, _CH, _K)
    dst3 = dst.reshape(_NW, _CH, _K)
    seg8 = jnp.tile(segment_ids[None, :], (8, 1))
    b8 = jnp.tile(b_o[None, :], (8, 1))

    gather, scatter = _sc_kernels()

    a = _mm_tc(x, W_i[:_ATOM])
    g0 = gather(a, src3)
    inp, m = _init_tc(g0, e, W_i[_ATOM:])
    for _ in range(4):
        parts = scatter(m, dst3)
        es = _combine_tc(parts)
        g = gather(es, src3)
        m = _update_tc(inp, g, m, W_h)
    parts = scatter(m, dst3)
    out = _final_tc(x, parts, seg8, W_o[:_ATOM], W_o[_ATOM:], b8, W_mlp)
    return out[:100]


# speedup vs baseline: 3.7747x; 1.4253x over previous
import functools

import jax
import jax.numpy as jnp
from jax import lax
from jax.experimental import pallas as pl
from jax.experimental.pallas import tpu as pltpu
from jax.experimental.pallas import tpu_sc as plsc

F32 = jnp.float32

_N = 10000
_E = 320000
_D = 128
_ATOM = 128
_NHALF = _N // 2

_NC = 2
_NS = 16
_NW = _NC * _NS
_PER_W = _E // _NW
_K = 80
_CH = _PER_W // _K
_NP = 10240
_RPT = _NP // _NS

_BLK = 2560
_NBLK = _E // _BLK


_GNBUF = 8
_GQ = 4
_SNBUF = 3
_SQ = 2


@functools.cache
def _sc_kernels():
    mesh = plsc.VectorSubcoreMesh(
        core_axis_name="c", subcore_axis_name="s", num_cores=_NC,
        num_subcores=_NS)

    @functools.partial(
        pl.kernel,
        out_type=jax.ShapeDtypeStruct((_E, _D), F32),
        mesh=mesh,
        scratch_types=[
            pltpu.VMEM((_CH, _K), jnp.int32),
            pltpu.VMEM((_GNBUF, _K, _D), F32),
            pltpu.SemaphoreType.DMA((_GNBUF,)),
            pltpu.SemaphoreType.DMA((_GNBUF,)),
        ],
    )
    def gather(tab_hbm, idx_hbm, out_hbm, idx_v, bufs, in_sems, out_sems):
        cid = lax.axis_index("c")
        sid = lax.axis_index("s")
        wid = sid * _NC + cid
        pltpu.sync_copy(idx_hbm.at[wid], idx_v)

        def in_desc(ch):
            b = ch % _GNBUF
            return pltpu.make_async_copy(
                tab_hbm.at[idx_v.at[ch]], bufs.at[b], in_sems.at[b])

        def out_desc(ch):
            b = ch % _GNBUF
            base = wid * _PER_W + ch * _K
            return pltpu.make_async_copy(
                bufs.at[b], out_hbm.at[pl.ds(base, _K)], out_sems.at[b])

        def body(ch, c):
            @pl.when(ch >= _GNBUF)
            def _():
                out_desc(ch - _GNBUF).wait()
            in_desc(ch).start()

            @pl.when(ch >= _GQ)
            def _():
                in_desc(ch - _GQ).wait()
                out_desc(ch - _GQ).start()
            return c

        lax.fori_loop(0, _CH, body, 0)

        def tail1(i, c):
            ch = _CH - _GQ + i
            in_desc(ch).wait()
            out_desc(ch).start()
            return c

        lax.fori_loop(0, _GQ, tail1, 0)

        def tail2(i, c):
            out_desc(_CH - _GNBUF + i).wait()
            return c

        lax.fori_loop(0, _GNBUF, tail2, 0)

    @functools.partial(
        pl.kernel,
        out_type=jax.ShapeDtypeStruct((_NC, _NP, _D), F32),
        mesh=mesh,
        scratch_types=[
            pltpu.VMEM((_CH, _K), jnp.int32),
            pltpu.VMEM((_SNBUF, _K, _D), F32),
            pltpu.VMEM_SHARED((_NP, _D), F32),
            pltpu.SemaphoreType.DMA((_SNBUF,)),
            pltpu.SemaphoreType.DMA((_SNBUF,)),
        ],
    )
    def scatter(msg_hbm, dst_hbm, out_hbm, idx_v, bufs, acc,
                in_sems, add_sems):
        cid = lax.axis_index("c")
        sid = lax.axis_index("s")
        wid = sid * _NC + cid
        z16 = jnp.zeros((16,), F32)

        def zrow(i, c):
            for j in range(8):
                bufs[0, i, pl.ds(j * 16, 16)] = z16
            return c

        lax.fori_loop(0, _K, zrow, 0)

        def zacc(k, c):
            pltpu.sync_copy(bufs.at[0], acc.at[pl.ds(sid * _RPT + k * _K, _K)])
            return c

        lax.fori_loop(0, _RPT // _K, zacc, 0)
        pltpu.sync_copy(dst_hbm.at[wid], idx_v)
        plsc.subcore_barrier()

        def in_desc(ch):
            b = ch % _SNBUF
            base = wid * _PER_W + ch * _K
            return pltpu.make_async_copy(
                msg_hbm.at[pl.ds(base, _K)], bufs.at[b], in_sems.at[b])

        def add_start(ch):
            b = ch % _SNBUF
            pltpu.async_copy(
                bufs.at[b], acc.at[idx_v.at[ch]], add_sems.at[b], add=True)

        def add_wait(ch):
            b = ch % _SNBUF
            pltpu.make_async_copy(
                bufs.at[b], acc.at[idx_v.at[ch]], add_sems.at[b]).wait()

        def body(ch, c):
            @pl.when(ch >= _SNBUF)
            def _():
                add_wait(ch - _SNBUF)
            in_desc(ch).start()

            @pl.when(ch >= _SQ)
            def _():
                in_desc(ch - _SQ).wait()
                add_start(ch - _SQ)
            return c

        lax.fori_loop(0, _CH, body, 0)

        def tail1(i, c):
            ch = _CH - _SQ + i
            in_desc(ch).wait()
            add_start(ch)
            return c

        lax.fori_loop(0, _SQ, tail1, 0)

        def tail2(i, c):
            add_wait(_CH - _SNBUF + i)
            return c

        lax.fori_loop(0, _SNBUF, tail2, 0)
        plsc.subcore_barrier()

        def drain(k, c):
            r = sid * _RPT + k * _K
            pltpu.sync_copy(acc.at[pl.ds(r, _K)], out_hbm.at[cid, pl.ds(r, _K)])
            return c

        lax.fori_loop(0, _RPT // _K, drain, 0)

    return gather, scatter


def _mm_tc(xx, ww):
    nb = 10

    def kk(x_ref, w_ref, o_ref):
        o_ref[...] = jnp.dot(x_ref[...], w_ref[...],
                             preferred_element_type=F32)

    return pl.pallas_call(
        kk,
        grid=(nb,),
        in_specs=[pl.BlockSpec((_N // nb, _D), lambda i: (i, 0)),
                  pl.BlockSpec((_D, _D), lambda i: (0, 0))],
        out_specs=pl.BlockSpec((_N // nb, _D), lambda i: (i, 0)),
        out_shape=jax.ShapeDtypeStruct((_N, _D), F32),
    )(xx, ww)


def _combine_tc(parts):
    nb = 10

    def kk(p_ref, o_ref):
        o_ref[...] = p_ref[0] + p_ref[1]

    return pl.pallas_call(
        kk,
        grid=(nb,),
        in_specs=[pl.BlockSpec((2, _NP // nb, _D), lambda i: (0, i, 0))],
        out_specs=pl.BlockSpec((_NP // nb, _D), lambda i: (i, 0)),
        out_shape=jax.ShapeDtypeStruct((_NP, _D), F32),
    )(parts)


def _init_tc(g0, e, wib):
    def kk(g_ref, e_ref, w_ref, inp_ref, m_ref):
        v = g_ref[...] + jnp.dot(e_ref[...], w_ref[...],
                                 preferred_element_type=F32)
        inp_ref[...] = v
        m_ref[...] = jnp.maximum(v, 0.0)

    return pl.pallas_call(
        kk,
        grid=(_NBLK,),
        in_specs=[pl.BlockSpec((_BLK, _D), lambda i: (i, 0)),
                  pl.BlockSpec((_BLK, 16), lambda i: (i, 0)),
                  pl.BlockSpec((16, _D), lambda i: (0, 0))],
        out_specs=[pl.BlockSpec((_BLK, _D), lambda i: (i, 0)),
                   pl.BlockSpec((_BLK, _D), lambda i: (i, 0))],
        out_shape=[jax.ShapeDtypeStruct((_E, _D), F32),
                   jax.ShapeDtypeStruct((_E, _D), F32)],
    )(g0, e, wib)


def _update_tc(inp, g, m, wh):
    def kk(inp_ref, g_ref, m_ref, w_ref, o_ref):
        mm = m_ref[...]
        up = jnp.roll(mm, -1, axis=0)
        dn = jnp.roll(mm, 1, axis=0)
        ridx = lax.broadcasted_iota(jnp.int32, (_BLK, _D), 0)
        msw = jnp.where(ridx % 2 == 0, up, dn)
        a = jnp.dot(g_ref[...] - msw, w_ref[...], preferred_element_type=F32)
        o_ref[...] = jnp.maximum(inp_ref[...] + a, 0.0)

    return pl.pallas_call(
        kk,
        grid=(_NBLK,),
        in_specs=[pl.BlockSpec((_BLK, _D), lambda i: (i, 0)),
                  pl.BlockSpec((_BLK, _D), lambda i: (i, 0)),
                  pl.BlockSpec((_BLK, _D), lambda i: (i, 0)),
                  pl.BlockSpec((_D, _D), lambda i: (0, 0))],
        out_specs=pl.BlockSpec((_BLK, _D), lambda i: (i, 0)),
        out_shape=jax.ShapeDtypeStruct((_E, _D), F32),
    )(inp, g, m, wh)


def _final_tc(x, parts, seg8, wo1, wo2, b8, wmlp):
    def kk(x_ref, p_ref, s_ref, wo1_ref, wo2_ref, b_ref, wm_ref, o_ref):
        es = p_ref[0, :_N, :] + p_ref[1, :_N, :]
        h = (jnp.dot(x_ref[...], wo1_ref[...], preferred_element_type=F32)
             + jnp.dot(es, wo2_ref[...], preferred_element_type=F32)
             + b_ref[0:1, :])
        h = jnp.maximum(h, 0.0)
        diff = h[_NHALF:, :] - h[:_NHALF, :]
        t = jnp.maximum(jnp.dot(diff, wm_ref[...],
                                preferred_element_type=F32), 0.0)
        seg = jnp.broadcast_to(s_ref[0:1, :], (128, _NHALF))
        oh = (seg == lax.broadcasted_iota(jnp.int32, (128, _NHALF), 0))
        o_ref[...] = jnp.dot(oh.astype(F32), t, preferred_element_type=F32)

    return pl.pallas_call(
        kk,
        out_shape=jax.ShapeDtypeStruct((128, _D), F32),
    )(x, parts, seg8, wo1, wo2, b8, wmlp)


def kernel(x, e, edge_index, segment_ids, W_i, W_h, W_o, b_o, W_mlp):
    src = edge_index[0]
    dst = edge_index[1]
    src3 = src.reshape(_NW, _CH, _K)
    dst3 = dst.reshape(_NW, _CH, _K)
    seg8 = jnp.tile(segment_ids[None, :], (8, 1))
    b8 = jnp.tile(b_o[None, :], (8, 1))

    gather, scatter = _sc_kernels()

    a = _mm_tc(x, W_i[:_ATOM])
    g0 = gather(a, src3)
    inp, m = _init_tc(g0, e, W_i[_ATOM:])
    for _ in range(4):
        parts = scatter(m, dst3)
        es = _combine_tc(parts)
        g = gather(es, src3)
        m = _update_tc(inp, g, m, W_h)
    parts = scatter(m, dst3)
    out = _final_tc(x, parts, seg8, W_o[:_ATOM], W_o[_ATOM:], b8, W_mlp)
    return out[:100]
